# baseline (speedup 1.0000x reference)
import functools

import jax
import jax.numpy as jnp
from jax import lax
from jax.experimental import pallas as pl
from jax.experimental.pallas import tpu as pltpu
from jax.experimental.pallas import tpu_sc as plsc

N = 4194304
VOCAB = 100
TPAD = 112

NC, NS, L = 2, 16, 16
NW = NC * NS
PER_W = N // NW
CHUNK = 8192
NBUF = 4
NCHUNK = PER_W // CHUNK
NGROUP = NCHUNK // NBUF


def _scale_shift_body(inp_hbm, z_hbm, scale_hbm, shift_hbm, out_hbm,
                      pair_v, stage_sc, stage_sh,
                      z0, z1, z2, z3, x0, x1, x2, x3, o0, o1, o2, o3,
                      si0, si1, si2, si3, so0, so1, so2, so3, st_sem):
    zb, xb, ob = (z0, z1, z2, z3), (x0, x1, x2, x3), (o0, o1, o2, o3)
    sin, sout = (si0, si1, si2, si3), (so0, so1, so2, so3)

    wid = lax.axis_index("s") * NC + lax.axis_index("c")
    base = wid * PER_W

    for b in range(NBUF):
        off = base + b * CHUNK
        pltpu.async_copy(z_hbm.at[pl.ds(off, CHUNK)], zb[b], sin[b])
        pltpu.async_copy(inp_hbm.at[pl.ds(off, CHUNK)], xb[b], sin[b])

    dsc = pltpu.async_copy(scale_hbm, stage_sc.at[pl.ds(0, VOCAB)], st_sem)
    dsh = pltpu.async_copy(shift_hbm, stage_sh.at[pl.ds(0, VOCAB)], st_sem)
    dsc.wait()
    dsh.wait()
    hi = jnp.full((L,), -65536, dtype=jnp.int32)
    lo = jnp.full((L,), 65535, dtype=jnp.int32)
    for t in range(TPAD // L):
        ts = pl.ds(t * L, L)
        u = plsc.bitcast(stage_sc[ts], jnp.int32)
        v = plsc.bitcast(stage_sh[ts], jnp.int32)
        u = (u + 32767 + ((u >> 16) & 1)) & hi
        v = ((v + 32767 + ((v >> 16) & 1)) >> 16) & lo
        pair_v[ts] = u | v

    def group(g, carry):
        for b in range(NBUF):
            off = base + (g * NBUF + b) * CHUNK
            pltpu.make_async_copy(z_hbm.at[pl.ds(off, CHUNK)], zb[b],
                                  sin[b]).wait()
            pltpu.make_async_copy(inp_hbm.at[pl.ds(off, CHUNK)], xb[b],
                                  sin[b]).wait()

            @pl.when(g > 0)
            def _wait_prev_out(off=off, b=b):
                poff = off - NBUF * CHUNK
                pltpu.make_async_copy(ob[b], out_hbm.at[pl.ds(poff, CHUNK)],
                                      sout[b]).wait()

            z_v, x_v, o_v = zb[b], xb[b], ob[b]

            @plsc.parallel_loop(0, CHUNK // L, unroll=4)
            def _compute(i, z_v=z_v, x_v=x_v, o_v=o_v):
                s = pl.ds(i * L, L)
                idx = z_v[s]
                pair = plsc.load_gather(pair_v, [idx])
                sc = plsc.bitcast(pair & hi, jnp.float32)
                sh = plsc.bitcast(pair << 16, jnp.float32)
                o_v[s] = x_v[s] * sc + sh

            @pl.when(g < NGROUP - 1)
            def _start_next_in(off=off, b=b):
                noff = off + NBUF * CHUNK
                pltpu.async_copy(z_hbm.at[pl.ds(noff, CHUNK)], zb[b], sin[b])
                pltpu.async_copy(inp_hbm.at[pl.ds(noff, CHUNK)], xb[b], sin[b])

            pltpu.async_copy(o_v, out_hbm.at[pl.ds(off, CHUNK)], sout[b])
        return carry

    lax.fori_loop(0, NGROUP, group, 0)

    for b in range(NBUF):
        off = base + ((NGROUP - 1) * NBUF + b) * CHUNK
        pltpu.make_async_copy(ob[b], out_hbm.at[pl.ds(off, CHUNK)],
                              sout[b]).wait()


@jax.jit
def kernel(input, z, scale_table, shift_table):
    inp_flat = input.reshape(N)
    z_i32 = z.astype(jnp.int32)
    scale_flat = scale_table.reshape(VOCAB)
    shift_flat = shift_table.reshape(VOCAB)

    mesh = plsc.VectorSubcoreMesh(core_axis_name="c", subcore_axis_name="s")
    run = functools.partial(
        pl.kernel,
        mesh=mesh,
        compiler_params=pltpu.CompilerParams(
            needs_layout_passes=False,
            disable_bounds_checks=True,
            disable_semaphore_checks=True,
            skip_device_barrier=True),
        out_type=jax.ShapeDtypeStruct((N,), jnp.float32),
        scratch_types=[
            pltpu.VMEM((TPAD,), jnp.int32),
            pltpu.VMEM((TPAD,), jnp.float32),
            pltpu.VMEM((TPAD,), jnp.float32),
            pltpu.VMEM((CHUNK,), jnp.int32),
            pltpu.VMEM((CHUNK,), jnp.int32),
            pltpu.VMEM((CHUNK,), jnp.int32),
            pltpu.VMEM((CHUNK,), jnp.int32),
            pltpu.VMEM((CHUNK,), jnp.float32),
            pltpu.VMEM((CHUNK,), jnp.float32),
            pltpu.VMEM((CHUNK,), jnp.float32),
            pltpu.VMEM((CHUNK,), jnp.float32),
            pltpu.VMEM((CHUNK,), jnp.float32),
            pltpu.VMEM((CHUNK,), jnp.float32),
            pltpu.VMEM((CHUNK,), jnp.float32),
            pltpu.VMEM((CHUNK,), jnp.float32),
            pltpu.SemaphoreType.DMA,
            pltpu.SemaphoreType.DMA,
            pltpu.SemaphoreType.DMA,
            pltpu.SemaphoreType.DMA,
            pltpu.SemaphoreType.DMA,
            pltpu.SemaphoreType.DMA,
            pltpu.SemaphoreType.DMA,
            pltpu.SemaphoreType.DMA,
            pltpu.SemaphoreType.DMA,
        ],
    )(_scale_shift_body)
    out_flat = run(inp_flat, z_i32, scale_flat, shift_flat)
    return out_flat.reshape(N, 1)


# speedup vs baseline: 1.0137x; 1.0137x over previous
import functools

import jax
import jax.numpy as jnp
from jax import lax
from jax.experimental import pallas as pl
from jax.experimental.pallas import tpu as pltpu
from jax.experimental.pallas import tpu_sc as plsc

N = 4194304
VOCAB = 100
TPAD = 112

NC, NS, L = 2, 16, 16
NW = NC * NS
PER_W = N // NW
CHUNK = 8192
NBUF = 4
NCHUNK = PER_W // CHUNK
NGROUP = NCHUNK // NBUF


def _scale_shift_body(inp_hbm, z_hbm, scale_hbm, shift_hbm, out_hbm,
                      pair_v, stage_sc, stage_sh,
                      z0, z1, z2, z3, x0, x1, x2, x3, o0, o1, o2, o3,
                      si0, si1, si2, si3, so0, so1, so2, so3, st_sem):
    zb, xb, ob = (z0, z1, z2, z3), (x0, x1, x2, x3), (o0, o1, o2, o3)
    sin, sout = (si0, si1, si2, si3), (so0, so1, so2, so3)

    wid = lax.axis_index("s") * NC + lax.axis_index("c")
    base = wid * PER_W

    for b in range(NBUF):
        off = base + b * CHUNK
        pltpu.async_copy(z_hbm.at[pl.ds(off, CHUNK)], zb[b], sin[b])
        pltpu.async_copy(inp_hbm.at[pl.ds(off, CHUNK)], xb[b], sin[b])

    dsc = pltpu.async_copy(scale_hbm, stage_sc.at[pl.ds(0, VOCAB)], st_sem)
    dsh = pltpu.async_copy(shift_hbm, stage_sh.at[pl.ds(0, VOCAB)], st_sem)
    dsc.wait()
    dsh.wait()
    hi = jnp.full((L,), -65536, dtype=jnp.int32)
    lo = jnp.full((L,), 65535, dtype=jnp.int32)
    for t in range(TPAD // L):
        ts = pl.ds(t * L, L)
        u = plsc.bitcast(stage_sc[ts], jnp.int32)
        v = plsc.bitcast(stage_sh[ts], jnp.int32)
        u = (u + 32767 + ((u >> 16) & 1)) & hi
        v = ((v + 32767 + ((v >> 16) & 1)) >> 16) & lo
        pair_v[ts] = u | v

    def group(g, carry):
        for b in range(NBUF):
            off = base + (g * NBUF + b) * CHUNK
            pltpu.make_async_copy(z_hbm.at[pl.ds(off, CHUNK)], zb[b],
                                  sin[b]).wait()
            pltpu.make_async_copy(inp_hbm.at[pl.ds(off, CHUNK)], xb[b],
                                  sin[b]).wait()

            @pl.when(g > 0)
            def _wait_prev_out(off=off, b=b):
                poff = off - NBUF * CHUNK
                pltpu.make_async_copy(ob[b], out_hbm.at[pl.ds(poff, CHUNK)],
                                      sout[b]).wait()

            z_v, x_v, o_v = zb[b], xb[b], ob[b]

            @plsc.parallel_loop(0, CHUNK // L, unroll=8)
            def _compute(i, z_v=z_v, x_v=x_v, o_v=o_v):
                s = pl.ds(i * L, L)
                idx = z_v[s]
                pair = plsc.load_gather(pair_v, [idx])
                sc = plsc.bitcast(pair & hi, jnp.float32)
                sh = plsc.bitcast(pair << 16, jnp.float32)
                o_v[s] = x_v[s] * sc + sh

            pltpu.async_copy(o_v, out_hbm.at[pl.ds(off, CHUNK)], sout[b])

            @pl.when(g < NGROUP - 1)
            def _start_next_in(off=off, b=b):
                noff = off + NBUF * CHUNK
                pltpu.async_copy(z_hbm.at[pl.ds(noff, CHUNK)], zb[b], sin[b])
                pltpu.async_copy(inp_hbm.at[pl.ds(noff, CHUNK)], xb[b], sin[b])
        return carry

    lax.fori_loop(0, NGROUP, group, 0)

    for b in range(NBUF):
        off = base + ((NGROUP - 1) * NBUF + b) * CHUNK
        pltpu.make_async_copy(ob[b], out_hbm.at[pl.ds(off, CHUNK)],
                              sout[b]).wait()


@jax.jit
def kernel(input, z, scale_table, shift_table):
    inp_flat = input.reshape(N)
    z_i32 = z.astype(jnp.int32)
    scale_flat = scale_table.reshape(VOCAB)
    shift_flat = shift_table.reshape(VOCAB)

    mesh = plsc.VectorSubcoreMesh(core_axis_name="c", subcore_axis_name="s")
    run = functools.partial(
        pl.kernel,
        mesh=mesh,
        compiler_params=pltpu.CompilerParams(
            needs_layout_passes=False,
            disable_bounds_checks=True,
            disable_semaphore_checks=True,
            skip_device_barrier=True),
        out_type=jax.ShapeDtypeStruct((N,), jnp.float32),
        scratch_types=[
            pltpu.VMEM((TPAD,), jnp.int32),
            pltpu.VMEM((TPAD,), jnp.float32),
            pltpu.VMEM((TPAD,), jnp.float32),
            pltpu.VMEM((CHUNK,), jnp.int32),
            pltpu.VMEM((CHUNK,), jnp.int32),
            pltpu.VMEM((CHUNK,), jnp.int32),
            pltpu.VMEM((CHUNK,), jnp.int32),
            pltpu.VMEM((CHUNK,), jnp.float32),
            pltpu.VMEM((CHUNK,), jnp.float32),
            pltpu.VMEM((CHUNK,), jnp.float32),
            pltpu.VMEM((CHUNK,), jnp.float32),
            pltpu.VMEM((CHUNK,), jnp.float32),
            pltpu.VMEM((CHUNK,), jnp.float32),
            pltpu.VMEM((CHUNK,), jnp.float32),
            pltpu.VMEM((CHUNK,), jnp.float32),
            pltpu.SemaphoreType.DMA,
            pltpu.SemaphoreType.DMA,
            pltpu.SemaphoreType.DMA,
            pltpu.SemaphoreType.DMA,
            pltpu.SemaphoreType.DMA,
            pltpu.SemaphoreType.DMA,
            pltpu.SemaphoreType.DMA,
            pltpu.SemaphoreType.DMA,
            pltpu.SemaphoreType.DMA,
        ],
    )(_scale_shift_body)
    out_flat = run(inp_flat, z_i32, scale_flat, shift_flat)
    return out_flat.reshape(N, 1)
